# baseline (speedup 1.0000x reference)
import jax
import jax.numpy as jnp
from jax.experimental import pallas as pl
from jax.experimental.pallas import tpu as pltpu

N = 10000
H = 64
BR = 400
BR_S = 2000

_ARB = pltpu.CompilerParams(dimension_semantics=("arbitrary",))
BF16 = jnp.bfloat16


def _rsqrt_f32(x):
    r = jax.lax.rsqrt(x)
    return r * (1.5 - 0.5 * x * r * r)


def _deg_body(a_ref, dinv_ref):
    deg = jnp.maximum(jnp.sum(a_ref[...], axis=1, keepdims=True), 1.0)
    dinv_ref[...] = _rsqrt_f32(deg)


def _encode_body(obs_ref, wall_ref, bd_ref, bs_ref, xs_ref):
    i = pl.program_id(0)
    z = jnp.dot(obs_ref[...].astype(BF16), wall_ref[...],
                preferred_element_type=jnp.float32)
    rows = jax.lax.broadcasted_iota(jnp.int32, (BR_S, 1), 0) + i * BR_S
    bias = jnp.where(rows == (N - 1), bs_ref[...], bd_ref[...])
    xs_ref[...] = jnp.maximum(z + bias, 0.0).astype(BF16)


def _norm_l1_body(a_ref, dinvc_ref, dinvr_ref, xs_ref, w_ref, b_ref,
                  n16_ref, h_ref, xsn_ref):
    n16 = ((dinvc_ref[...] * a_ref[...]) * dinvr_ref[...]).astype(BF16)
    n16_ref[...] = n16
    u = jnp.dot(n16, xs_ref[...], preferred_element_type=jnp.float32)
    h = jnp.maximum(
        jnp.dot(u.astype(BF16), w_ref[...],
                preferred_element_type=jnp.float32) + b_ref[...], 0.0)
    h_ref[...] = h
    xsn_ref[...] = h.astype(BF16)


def _layer_body(n16_ref, xs_ref, w_ref, b_ref, h_ref, xsn_ref, cs_ref):
    i = pl.program_id(0)
    u = jnp.dot(n16_ref[...], xs_ref[...], preferred_element_type=jnp.float32)
    h = jnp.maximum(
        jnp.dot(u.astype(BF16), w_ref[...],
                preferred_element_type=jnp.float32) + b_ref[...], 0.0)
    h_ref[...] = h
    xsn_ref[...] = h.astype(BF16)
    part = jnp.sum(h, axis=0, keepdims=True)

    @pl.when(i == 0)
    def _():
        cs_ref[...] = part

    @pl.when(i > 0)
    def _():
        cs_ref[...] += part


def _head_body(h_ref, mean_ref, srv_ref, wf1_ref, bf1_ref, wf2_ref, bf2_ref,
               out_ref):
    wf1 = wf1_ref[...]
    cmean = jnp.dot(mean_ref[...].astype(BF16), wf1[H:2 * H],
                    preferred_element_type=jnp.float32)
    csrv = jnp.dot(srv_ref[...].astype(BF16), wf1[2 * H:3 * H],
                   preferred_element_type=jnp.float32)
    t = (jnp.dot(h_ref[...].astype(BF16), wf1[0:H],
                 preferred_element_type=jnp.float32)
         + cmean + csrv + bf1_ref[...])
    t = jnp.maximum(t, 0.0)
    out_ref[...] = jnp.dot(t.astype(BF16), wf2_ref[...],
                           preferred_element_type=jnp.float32) + bf2_ref[...]


def _full(shape):
    return pl.BlockSpec(shape, lambda i: (0,) * len(shape))


def _rows(shape):
    return pl.BlockSpec(shape, lambda i: (i,) + (0,) * (len(shape) - 1))


def kernel(device_obs, server_obs, adjacency, W_dev, b_dev, W_srv, b_srv,
           W1, b1, W2, b2, W3, b3, Wf1, bf1, Wf2, bf2):
    f32 = jnp.float32
    n_dev = device_obs.shape[1]
    dev = device_obs.reshape(n_dev, device_obs.shape[2])

    dinv = pl.pallas_call(
        _deg_body,
        grid=(N // BR,),
        in_specs=[_rows((BR, N))],
        out_specs=_rows((BR, 1)),
        out_shape=jax.ShapeDtypeStruct((N, 1), f32),
        compiler_params=_ARB,
    )(adjacency)
    dinv_row = dinv.reshape(1, N)

    obs = jnp.concatenate(
        [jnp.pad(dev, ((0, 0), (0, server_obs.shape[1]))),
         jnp.pad(server_obs, ((0, 0), (dev.shape[1], 0)))], axis=0)
    w_all = jnp.concatenate([W_dev, W_srv], axis=0).astype(BF16)
    xs = pl.pallas_call(
        _encode_body,
        grid=(N // BR_S,),
        in_specs=[_rows((BR_S, obs.shape[1])), _full((obs.shape[1], H)),
                  _full((1, H)), _full((1, H))],
        out_specs=_rows((BR_S, H)),
        out_shape=jax.ShapeDtypeStruct((N, H), BF16),
        compiler_params=_ARB,
    )(obs, w_all, b_dev.reshape(1, H), b_srv.reshape(1, H))

    n16, h, xs = pl.pallas_call(
        _norm_l1_body,
        grid=(N // BR,),
        in_specs=[_rows((BR, N)), _rows((BR, 1)), _full((1, N)),
                  _full((N, H)), _full((H, H)), _full((1, H))],
        out_specs=[_rows((BR, N)), _rows((BR, H)), _rows((BR, H))],
        out_shape=[jax.ShapeDtypeStruct((N, N), BF16),
                   jax.ShapeDtypeStruct((N, H), f32),
                   jax.ShapeDtypeStruct((N, H), BF16)],
        compiler_params=_ARB,
    )(adjacency, dinv, dinv_row, xs, W1.astype(BF16), b1.reshape(1, H))

    colsum = None
    for W, b in ((W2, b2), (W3, b3)):
        h, xs, colsum = pl.pallas_call(
            _layer_body,
            grid=(N // BR,),
            in_specs=[_rows((BR, N)), _full((N, H)),
                      _full((H, H)), _full((1, H))],
            out_specs=[_rows((BR, H)), _rows((BR, H)), _full((1, H))],
            out_shape=[jax.ShapeDtypeStruct((N, H), f32),
                       jax.ShapeDtypeStruct((N, H), BF16),
                       jax.ShapeDtypeStruct((1, H), f32)],
            compiler_params=_ARB,
        )(n16, xs, W.astype(BF16), b.reshape(1, H))

    srv = jax.lax.slice(h, (N - 1, 0), (N, H))
    mean = (colsum - srv) / n_dev

    out = pl.pallas_call(
        _head_body,
        grid=(N // BR_S,),
        in_specs=[_rows((BR_S, H)), _full((1, H)), _full((1, H)),
                  _full(Wf1.shape), _full((1, Wf1.shape[1])),
                  _full(Wf2.shape), _full((1, 1))],
        out_specs=_rows((BR_S, 1)),
        out_shape=jax.ShapeDtypeStruct((N, 1), f32),
        compiler_params=_ARB,
    )(h, mean, srv, Wf1.astype(BF16), bf1.reshape(1, -1),
      Wf2.astype(BF16), bf2.reshape(1, 1))

    return out[:n_dev, 0].reshape(1, n_dev)


# speedup vs baseline: 1.0727x; 1.0727x over previous
import jax
import jax.numpy as jnp
from jax.experimental import pallas as pl
from jax.experimental.pallas import tpu as pltpu

N = 10000
H = 64
BR = 400
BR_S = 2000

_ARB = pltpu.CompilerParams(dimension_semantics=("arbitrary",))
BF16 = jnp.bfloat16


def _rsqrt_f32(x):
    r = jax.lax.rsqrt(x)
    return r * (1.5 - 0.5 * x * r * r)


def _hilo(v):
    hi = v.astype(BF16)
    lo = (v - hi.astype(jnp.float32)).astype(BF16)
    return jnp.concatenate([hi, lo], axis=1)


def _prep_body(a_ref, a16_ref, dinv_ref):
    a = a_ref[...]
    deg = jnp.maximum(jnp.sum(a, axis=1, keepdims=True), 1.0)
    dinv_ref[...] = _rsqrt_f32(deg)
    a16_ref[...] = a.astype(BF16)


def _encode_body(obs_ref, wall_ref, bd_ref, bs_ref, dinv_ref, xs_ref):
    i = pl.program_id(0)
    z = jnp.dot(obs_ref[...].astype(BF16), wall_ref[...],
                preferred_element_type=jnp.float32)
    rows = jax.lax.broadcasted_iota(jnp.int32, (BR_S, 1), 0) + i * BR_S
    bias = jnp.where(rows == (N - 1), bs_ref[...], bd_ref[...])
    t = jnp.maximum(z + bias, 0.0).astype(BF16)
    xs_ref[...] = _hilo(dinv_ref[...] * t.astype(jnp.float32))


def _layer_body(a16_ref, xs_ref, dinv_ref, w_ref, b_ref, h_ref, xsn_ref,
                cs_ref):
    i = pl.program_id(0)
    u2 = jnp.dot(a16_ref[...], xs_ref[...], preferred_element_type=jnp.float32)
    dinv = dinv_ref[...]
    u = (u2[:, :H] + u2[:, H:]) * dinv
    h = jnp.maximum(
        jnp.dot(u.astype(BF16), w_ref[...],
                preferred_element_type=jnp.float32) + b_ref[...], 0.0)
    h_ref[...] = h
    t = h.astype(BF16)
    xsn_ref[...] = _hilo(dinv * t.astype(jnp.float32))
    part = jnp.sum(h, axis=0, keepdims=True)

    @pl.when(i == 0)
    def _():
        cs_ref[...] = part

    @pl.when(i > 0)
    def _():
        cs_ref[...] += part


def _head_body(h_ref, mean_ref, srv_ref, wf1_ref, bf1_ref, wf2_ref, bf2_ref,
               out_ref):
    wf1 = wf1_ref[...]
    cmean = jnp.dot(mean_ref[...].astype(BF16), wf1[H:2 * H],
                    preferred_element_type=jnp.float32)
    csrv = jnp.dot(srv_ref[...].astype(BF16), wf1[2 * H:3 * H],
                   preferred_element_type=jnp.float32)
    t = (jnp.dot(h_ref[...].astype(BF16), wf1[0:H],
                 preferred_element_type=jnp.float32)
         + cmean + csrv + bf1_ref[...])
    t = jnp.maximum(t, 0.0)
    out_ref[...] = jnp.dot(t.astype(BF16), wf2_ref[...],
                           preferred_element_type=jnp.float32) + bf2_ref[...]


def _full(shape):
    return pl.BlockSpec(shape, lambda i: (0,) * len(shape))


def _rows(shape):
    return pl.BlockSpec(shape, lambda i: (i,) + (0,) * (len(shape) - 1))


def kernel(device_obs, server_obs, adjacency, W_dev, b_dev, W_srv, b_srv,
           W1, b1, W2, b2, W3, b3, Wf1, bf1, Wf2, bf2):
    f32 = jnp.float32
    n_dev = device_obs.shape[1]
    dev = device_obs.reshape(n_dev, device_obs.shape[2])

    a16, dinv = pl.pallas_call(
        _prep_body,
        grid=(N // BR,),
        in_specs=[_rows((BR, N))],
        out_specs=[_rows((BR, N)), _rows((BR, 1))],
        out_shape=[jax.ShapeDtypeStruct((N, N), BF16),
                   jax.ShapeDtypeStruct((N, 1), f32)],
        compiler_params=_ARB,
    )(adjacency)

    obs = jnp.concatenate(
        [jnp.pad(dev, ((0, 0), (0, server_obs.shape[1]))),
         jnp.pad(server_obs, ((0, 0), (dev.shape[1], 0)))], axis=0)
    w_all = jnp.concatenate([W_dev, W_srv], axis=0).astype(BF16)
    xs = pl.pallas_call(
        _encode_body,
        grid=(N // BR_S,),
        in_specs=[_rows((BR_S, obs.shape[1])), _full((obs.shape[1], H)),
                  _full((1, H)), _full((1, H)), _rows((BR_S, 1))],
        out_specs=_rows((BR_S, 2 * H)),
        out_shape=jax.ShapeDtypeStruct((N, 2 * H), BF16),
        compiler_params=_ARB,
    )(obs, w_all, b_dev.reshape(1, H), b_srv.reshape(1, H), dinv)

    h = None
    colsum = None
    for W, b in ((W1, b1), (W2, b2), (W3, b3)):
        h, xs, colsum = pl.pallas_call(
            _layer_body,
            grid=(N // BR,),
            in_specs=[_rows((BR, N)), _full((N, 2 * H)), _rows((BR, 1)),
                      _full((H, H)), _full((1, H))],
            out_specs=[_rows((BR, H)), _rows((BR, 2 * H)), _full((1, H))],
            out_shape=[jax.ShapeDtypeStruct((N, H), f32),
                       jax.ShapeDtypeStruct((N, 2 * H), BF16),
                       jax.ShapeDtypeStruct((1, H), f32)],
            compiler_params=_ARB,
        )(a16, xs, dinv, W.astype(BF16), b.reshape(1, H))

    srv = jax.lax.slice(h, (N - 1, 0), (N, H))
    mean = (colsum - srv) / n_dev

    out = pl.pallas_call(
        _head_body,
        grid=(N // BR_S,),
        in_specs=[_rows((BR_S, H)), _full((1, H)), _full((1, H)),
                  _full(Wf1.shape), _full((1, Wf1.shape[1])),
                  _full(Wf2.shape), _full((1, 1))],
        out_specs=_rows((BR_S, 1)),
        out_shape=jax.ShapeDtypeStruct((N, 1), f32),
        compiler_params=_ARB,
    )(h, mean, srv, Wf1.astype(BF16), bf1.reshape(1, -1),
      Wf2.astype(BF16), bf2.reshape(1, 1))

    return out[:n_dev, 0].reshape(1, n_dev)
